# baseline (speedup 1.0000x reference)
import functools

import jax
import jax.numpy as jnp
from jax import lax
from jax.experimental import pallas as pl
from jax.experimental.pallas import tpu as pltpu
from jax.experimental.pallas import tpu_sc as plsc

B = 16384
F = 26
FIELD = 40000
TOTAL = F * FIELD
NW = 32
RPW = B // NW
EPW = RPW * F
L = 16

FB = (0, 2, 5, 9, 14, 20, 26)
FILL = (0, 81920, 200704, 360448, 561152, 800768, TOTAL)
NS = 6


def _stage_chunks(s, sid_is_last):
    lo, hi = FILL[s], FILL[s + 1]
    n = hi - lo
    per = (n // 16) // 128 * 128
    if per * 16 == n:
        return lo, per, per
    per_hi = -((-n // 16) // 128) * 128
    return lo, per_hi, n - 15 * per_hi


def _sc_kernel(xt_hbm, table_hbm, bias_hbm, out_hbm,
               idx_v, vals_v, bias_v, out_v, stab,
               xsem, bsem, fsems, gsems):
    cid = lax.axis_index("c")
    sid = lax.axis_index("s")
    wid = sid * 2 + cid
    base = wid * RPW

    for f in range(F):
        pltpu.async_copy(xt_hbm.at[f, pl.ds(base, RPW)],
                         idx_v.at[pl.ds(f * RPW, RPW)], xsem)
    pltpu.async_copy(bias_hbm, bias_v, bsem)

    for s in range(NS):
        lo, per, last = _stage_chunks(s, None)
        if per == last:
            pltpu.async_copy(table_hbm.at[0, pl.ds(lo + sid * per, per)],
                             stab.at[pl.ds(lo + sid * per, per)], fsems[s])
        else:
            @pl.when(sid < 15)
            def _(lo=lo, per=per, s=s):
                pltpu.async_copy(table_hbm.at[0, pl.ds(lo + sid * per, per)],
                                 stab.at[pl.ds(lo + sid * per, per)], fsems[s])

            @pl.when(sid == 15)
            def _(lo=lo, per=per, last=last, s=s):
                pltpu.async_copy(
                    table_hbm.at[0, pl.ds(lo + 15 * per, last)],
                    stab.at[pl.ds(lo + 15 * per, last)], fsems[s])

    pltpu.make_async_copy(xt_hbm.at[0, pl.ds(0, EPW)], idx_v, xsem).wait()

    with jax.named_scope("idx_add"):
        def idx_body(c, carry):
            f = c >> 2
            off = f * FIELD
            p = c * 128
            for k in range(8):
                q = p + k * L
                idx_v[pl.ds(q, L)] = idx_v[pl.ds(q, L)] + off
            return carry
        lax.fori_loop(0, EPW // 128, idx_body, 0)

    pltpu.make_async_copy(bias_hbm, bias_v, bsem).wait()
    bias_vec = bias_v[...]

    for s in range(NS):
        lo, per, last = _stage_chunks(s, None)
        with jax.named_scope(f"fill{s}_wait"):
            if per == last:
                pltpu.make_async_copy(
                    table_hbm.at[0, pl.ds(lo + sid * per, per)],
                    stab.at[pl.ds(lo + sid * per, per)], fsems[s]).wait()
            else:
                @pl.when(sid < 15)
                def _(lo=lo, per=per, s=s):
                    pltpu.make_async_copy(
                        table_hbm.at[0, pl.ds(lo + sid * per, per)],
                        stab.at[pl.ds(lo + sid * per, per)], fsems[s]).wait()

                @pl.when(sid == 15)
                def _(lo=lo, per=per, last=last, s=s):
                    pltpu.make_async_copy(
                        table_hbm.at[0, pl.ds(lo + 15 * per, last)],
                        stab.at[pl.ds(lo + 15 * per, last)], fsems[s]).wait()
            plsc.subcore_barrier()

        e0, e1 = FB[s] * RPW, FB[s + 1] * RPW
        with jax.named_scope(f"gather{s}_start"):
            pltpu.async_copy(stab.at[idx_v.at[pl.ds(e0, e1 - e0)]],
                             vals_v.at[pl.ds(e0, e1 - e0)], gsems[s])

        if s > 0:
            _drain_reduce(s - 1, bias_vec, idx_v, vals_v, out_v, stab, gsems)
    _drain_reduce(NS - 1, bias_vec, idx_v, vals_v, out_v, stab, gsems)

    pltpu.sync_copy(out_v, out_hbm.at[pl.ds(base, RPW)])


def _drain_reduce(s, bias_vec, idx_v, vals_v, out_v, stab, gsems):
    e0, e1 = FB[s] * RPW, FB[s + 1] * RPW
    with jax.named_scope(f"gather{s}_wait"):
        pltpu.make_async_copy(stab.at[idx_v.at[pl.ds(e0, e1 - e0)]],
                              vals_v.at[pl.ds(e0, e1 - e0)], gsems[s]).wait()

    with jax.named_scope(f"reduce{s}"):
        def red_body(j, carry):
            p = j * L
            acc = bias_vec if s == 0 else out_v[pl.ds(p, L)]
            for f in range(FB[s], FB[s + 1]):
                acc = acc + vals_v[pl.ds(f * RPW + p, L)]
            out_v[pl.ds(p, L)] = acc
            return carry
        lax.fori_loop(0, RPW // L, red_body, 0)


@jax.jit
def _features_linear(xt, tablet, bias16):
    mesh = plsc.VectorSubcoreMesh(core_axis_name="c", subcore_axis_name="s")
    run = functools.partial(
        pl.kernel,
        mesh=mesh,
        compiler_params=pltpu.CompilerParams(needs_layout_passes=False),
        out_type=jax.ShapeDtypeStruct((B,), jnp.float32),
        scratch_types=[
            pltpu.VMEM((EPW,), jnp.int32),
            pltpu.VMEM((EPW,), jnp.float32),
            pltpu.VMEM((L,), jnp.float32),
            pltpu.VMEM((RPW,), jnp.float32),
            pltpu.VMEM_SHARED((TOTAL,), jnp.float32),
            pltpu.SemaphoreType.DMA,
            pltpu.SemaphoreType.DMA,
            [pltpu.SemaphoreType.DMA] * NS,
            [pltpu.SemaphoreType.DMA] * NS,
        ],
    )(_sc_kernel)
    return run(xt, tablet, bias16)


def kernel(x, table, bias):
    xt = x.T.astype(jnp.int32)
    bias16 = jnp.broadcast_to(bias.astype(jnp.float32), (L,))
    out = _features_linear(xt, table.T, bias16)
    return out.reshape(B, 1)


# speedup vs baseline: 1.0122x; 1.0122x over previous
import functools

import jax
import jax.numpy as jnp
from jax import lax
from jax.experimental import pallas as pl
from jax.experimental.pallas import tpu as pltpu
from jax.experimental.pallas import tpu_sc as plsc

B = 16384
F = 26
FIELD = 40000
TOTAL = F * FIELD
NW = 32
RPW = B // NW
EPW = RPW * F
L = 16

FB = (0, 2, 7, 13, 20, 26)
FILL = (0, 81920, 280576, 520192, 800768, TOTAL)
NS = 5


def _stage_chunks(s, sid_is_last):
    lo, hi = FILL[s], FILL[s + 1]
    n = hi - lo
    per = (n // 16) // 128 * 128
    if per * 16 == n:
        return lo, per, per
    per_hi = -((-n // 16) // 128) * 128
    return lo, per_hi, n - 15 * per_hi


def _sc_kernel(xt_hbm, table_hbm, bias_hbm, out_hbm,
               idx_v, vals_v, bias_v, out_v, stab,
               xsem, bsem, fsems, gsems):
    cid = lax.axis_index("c")
    sid = lax.axis_index("s")
    wid = sid * 2 + cid
    base = wid * RPW

    for f in range(F):
        pltpu.async_copy(xt_hbm.at[f, pl.ds(base, RPW)],
                         idx_v.at[pl.ds(f * RPW, RPW)], xsem)
    pltpu.async_copy(bias_hbm, bias_v, bsem)

    for s in range(NS):
        lo, per, last = _stage_chunks(s, None)
        if per == last:
            pltpu.async_copy(table_hbm.at[0, pl.ds(lo + sid * per, per)],
                             stab.at[pl.ds(lo + sid * per, per)], fsems[s])
        else:
            @pl.when(sid < 15)
            def _(lo=lo, per=per, s=s):
                pltpu.async_copy(table_hbm.at[0, pl.ds(lo + sid * per, per)],
                                 stab.at[pl.ds(lo + sid * per, per)], fsems[s])

            @pl.when(sid == 15)
            def _(lo=lo, per=per, last=last, s=s):
                pltpu.async_copy(
                    table_hbm.at[0, pl.ds(lo + 15 * per, last)],
                    stab.at[pl.ds(lo + 15 * per, last)], fsems[s])

    pltpu.make_async_copy(xt_hbm.at[0, pl.ds(0, EPW)], idx_v, xsem).wait()

    with jax.named_scope("idx_add"):
        def idx_body(c, carry):
            f = c >> 2
            off = f * FIELD
            p = c * 128
            for k in range(8):
                q = p + k * L
                idx_v[pl.ds(q, L)] = idx_v[pl.ds(q, L)] + off
            return carry
        lax.fori_loop(0, EPW // 128, idx_body, 0)

    pltpu.make_async_copy(bias_hbm, bias_v, bsem).wait()
    bias_vec = bias_v[...]

    for s in range(NS):
        lo, per, last = _stage_chunks(s, None)
        with jax.named_scope(f"fill{s}_wait"):
            if per == last:
                pltpu.make_async_copy(
                    table_hbm.at[0, pl.ds(lo + sid * per, per)],
                    stab.at[pl.ds(lo + sid * per, per)], fsems[s]).wait()
            else:
                @pl.when(sid < 15)
                def _(lo=lo, per=per, s=s):
                    pltpu.make_async_copy(
                        table_hbm.at[0, pl.ds(lo + sid * per, per)],
                        stab.at[pl.ds(lo + sid * per, per)], fsems[s]).wait()

                @pl.when(sid == 15)
                def _(lo=lo, per=per, last=last, s=s):
                    pltpu.make_async_copy(
                        table_hbm.at[0, pl.ds(lo + 15 * per, last)],
                        stab.at[pl.ds(lo + 15 * per, last)], fsems[s]).wait()
            plsc.subcore_barrier()

        e0, e1 = FB[s] * RPW, FB[s + 1] * RPW
        with jax.named_scope(f"gather{s}_start"):
            pltpu.async_copy(stab.at[idx_v.at[pl.ds(e0, e1 - e0)]],
                             vals_v.at[pl.ds(e0, e1 - e0)], gsems[s])

        if s > 0:
            _drain_reduce(s - 1, bias_vec, idx_v, vals_v, out_v, stab, gsems)
    _drain_reduce(NS - 1, bias_vec, idx_v, vals_v, out_v, stab, gsems)

    pltpu.sync_copy(out_v, out_hbm.at[pl.ds(base, RPW)])


def _drain_reduce(s, bias_vec, idx_v, vals_v, out_v, stab, gsems):
    e0, e1 = FB[s] * RPW, FB[s + 1] * RPW
    with jax.named_scope(f"gather{s}_wait"):
        pltpu.make_async_copy(stab.at[idx_v.at[pl.ds(e0, e1 - e0)]],
                              vals_v.at[pl.ds(e0, e1 - e0)], gsems[s]).wait()

    with jax.named_scope(f"reduce{s}"):
        def red_body(j, carry):
            p = j * L
            acc = bias_vec if s == 0 else out_v[pl.ds(p, L)]
            for f in range(FB[s], FB[s + 1]):
                acc = acc + vals_v[pl.ds(f * RPW + p, L)]
            out_v[pl.ds(p, L)] = acc
            return carry
        lax.fori_loop(0, RPW // L, red_body, 0)


@jax.jit
def _features_linear(xt, tablet, bias16):
    mesh = plsc.VectorSubcoreMesh(core_axis_name="c", subcore_axis_name="s")
    run = functools.partial(
        pl.kernel,
        mesh=mesh,
        compiler_params=pltpu.CompilerParams(needs_layout_passes=False),
        out_type=jax.ShapeDtypeStruct((B,), jnp.float32),
        scratch_types=[
            pltpu.VMEM((EPW,), jnp.int32),
            pltpu.VMEM((EPW,), jnp.float32),
            pltpu.VMEM((L,), jnp.float32),
            pltpu.VMEM((RPW,), jnp.float32),
            pltpu.VMEM_SHARED((TOTAL,), jnp.float32),
            pltpu.SemaphoreType.DMA,
            pltpu.SemaphoreType.DMA,
            [pltpu.SemaphoreType.DMA] * NS,
            [pltpu.SemaphoreType.DMA] * NS,
        ],
    )(_sc_kernel)
    return run(xt, tablet, bias16)


def kernel(x, table, bias):
    xt = x.T.astype(jnp.int32)
    bias16 = jnp.broadcast_to(bias.astype(jnp.float32), (L,))
    out = _features_linear(xt, table.T, bias16)
    return out.reshape(B, 1)


# speedup vs baseline: 1.0159x; 1.0037x over previous
import functools

import jax
import jax.numpy as jnp
from jax import lax
from jax.experimental import pallas as pl
from jax.experimental.pallas import tpu as pltpu
from jax.experimental.pallas import tpu_sc as plsc

B = 16384
F = 26
FIELD = 40000
TOTAL = F * FIELD
NW = 32
RPW = B // NW
EPW = RPW * F
L = 16

FB = (0, 2, 7, 13, 20, 26)
FILL = (0, 81920, 280576, 520192, 800768, TOTAL)
NS = 5


def _stage_chunks(s):
    lo, hi = FILL[s], FILL[s + 1]
    n = hi - lo
    per = (n // 16) // 128 * 128
    if per * 16 == n:
        return lo, per, per
    per_hi = -((-n // 16) // 128) * 128
    return lo, per_hi, n - 15 * per_hi


def _sc_kernel(xt_hbm, table_hbm, bias_hbm, out_hbm,
               idx_v, vals_v, bias_v, out_v, stab,
               xsem, bsem, fsems, gsems):
    cid = lax.axis_index("c")
    sid = lax.axis_index("s")
    wid = sid * 2 + cid
    base = wid * RPW

    for f in range(F):
        pltpu.async_copy(xt_hbm.at[f, pl.ds(base, RPW)],
                         idx_v.at[pl.ds(f * RPW, RPW)], xsem)
    pltpu.async_copy(bias_hbm, bias_v, bsem)

    for s in range(NS):
        lo, per, last = _stage_chunks(s)
        if per == last:
            pltpu.async_copy(table_hbm.at[0, pl.ds(lo + sid * per, per)],
                             stab.at[pl.ds(lo + sid * per, per)], fsems[s])
        else:
            @pl.when(sid < 15)
            def _(lo=lo, per=per, s=s):
                pltpu.async_copy(table_hbm.at[0, pl.ds(lo + sid * per, per)],
                                 stab.at[pl.ds(lo + sid * per, per)], fsems[s])

            @pl.when(sid == 15)
            def _(lo=lo, per=per, last=last, s=s):
                pltpu.async_copy(
                    table_hbm.at[0, pl.ds(lo + 15 * per, last)],
                    stab.at[pl.ds(lo + 15 * per, last)], fsems[s])

    pltpu.make_async_copy(xt_hbm.at[0, pl.ds(0, EPW)], idx_v, xsem).wait()

    with jax.named_scope("idx_add"):
        def idx_body(c, carry):
            f = c >> 2
            off = f * FIELD
            p = c * 128
            for k in range(8):
                q = p + k * L
                idx_v[pl.ds(q, L)] = idx_v[pl.ds(q, L)] + off
            return carry
        lax.fori_loop(0, EPW // 128, idx_body, 0)

    pltpu.make_async_copy(bias_hbm, bias_v, bsem).wait()
    bias_vec = bias_v[...]

    for s in range(NS):
        lo, per, last = _stage_chunks(s)
        with jax.named_scope(f"fill{s}_wait"):
            if per == last:
                pltpu.make_async_copy(
                    table_hbm.at[0, pl.ds(lo + sid * per, per)],
                    stab.at[pl.ds(lo + sid * per, per)], fsems[s]).wait()
            else:
                @pl.when(sid < 15)
                def _(lo=lo, per=per, s=s):
                    pltpu.make_async_copy(
                        table_hbm.at[0, pl.ds(lo + sid * per, per)],
                        stab.at[pl.ds(lo + sid * per, per)], fsems[s]).wait()

                @pl.when(sid == 15)
                def _(lo=lo, per=per, last=last, s=s):
                    pltpu.make_async_copy(
                        table_hbm.at[0, pl.ds(lo + 15 * per, last)],
                        stab.at[pl.ds(lo + 15 * per, last)], fsems[s]).wait()
            plsc.subcore_barrier()

        e0, e1 = FB[s] * RPW, FB[s + 1] * RPW
        with jax.named_scope(f"gather{s}_start"):
            pltpu.async_copy(stab.at[idx_v.at[pl.ds(e0, e1 - e0)]],
                             vals_v.at[pl.ds(e0, e1 - e0)], gsems[s])

        if s > 0:
            _drain_reduce(s - 1, bias_vec, idx_v, vals_v, out_v, stab, gsems)
    _drain_reduce(NS - 1, bias_vec, idx_v, vals_v, out_v, stab, gsems)

    pltpu.sync_copy(out_v, out_hbm.at[pl.ds(base, RPW)])


def _drain_reduce(s, bias_vec, idx_v, vals_v, out_v, stab, gsems):
    e0, e1 = FB[s] * RPW, FB[s + 1] * RPW
    with jax.named_scope(f"gather{s}_wait"):
        pltpu.make_async_copy(stab.at[idx_v.at[pl.ds(e0, e1 - e0)]],
                              vals_v.at[pl.ds(e0, e1 - e0)], gsems[s]).wait()

    with jax.named_scope(f"reduce{s}"):
        def red_body(j, carry):
            p = j * L
            acc = bias_vec if s == 0 else out_v[pl.ds(p, L)]
            for f in range(FB[s], FB[s + 1]):
                acc = acc + vals_v[pl.ds(f * RPW + p, L)]
            out_v[pl.ds(p, L)] = acc
            return carry
        lax.fori_loop(0, RPW // L, red_body, 0)


@jax.jit
def _features_linear(xt, tablet, bias16):
    mesh = plsc.VectorSubcoreMesh(core_axis_name="c", subcore_axis_name="s")
    run = functools.partial(
        pl.kernel,
        mesh=mesh,
        compiler_params=pltpu.CompilerParams(needs_layout_passes=False),
        out_type=jax.ShapeDtypeStruct((B,), jnp.float32),
        scratch_types=[
            pltpu.VMEM((EPW,), jnp.int32),
            pltpu.VMEM((EPW,), jnp.float32),
            pltpu.VMEM((L,), jnp.float32),
            pltpu.VMEM((RPW,), jnp.float32),
            pltpu.VMEM_SHARED((TOTAL,), jnp.float32),
            pltpu.SemaphoreType.DMA,
            pltpu.SemaphoreType.DMA,
            [pltpu.SemaphoreType.DMA] * NS,
            [pltpu.SemaphoreType.DMA] * NS,
        ],
    )(_sc_kernel)
    return run(xt, tablet, bias16)


def kernel(x, table, bias):
    xt = x.T.astype(jnp.int32)
    bias16 = jnp.broadcast_to(bias.astype(jnp.float32), (L,))
    out = _features_linear(xt, table.T, bias16)
    return out.reshape(B, 1)
